# baseline (speedup 1.0000x reference)
import functools

import jax
import jax.numpy as jnp
from jax import lax
from jax.experimental import pallas as pl
from jax.experimental.pallas import tpu as pltpu
from jax.experimental.pallas import tpu_sc as plsc

_N = 10000
_E = 320000
_D = 128
_G = 128
_C = 10
_H3 = 3 * _D

_NC, _NS = 2, 16
_NW = _NC * _NS
_CH = 112
_CPT = 90
_NSL = 3
_NGR = _CPT // _NSL
_NPAD = 10240
_RPT = _NPAD // _NS
_ZB = 80

_RB = 1000
_NBLK = _N // _RB


def _pre_body(x_ref, w_ref, m_ref):
    m_ref[...] = jnp.dot(x_ref[...], w_ref[...],
                         preferred_element_type=jnp.float32)


def _pre(x, W):
    return pl.pallas_call(
        _pre_body,
        grid=(_NBLK,),
        in_specs=[
            pl.BlockSpec((_RB, _D), lambda i: (i, 0)),
            pl.BlockSpec((_D, _D), lambda i: (0, 0)),
        ],
        out_specs=pl.BlockSpec((_RB, _D), lambda i: (i, 0)),
        out_shape=jax.ShapeDtypeStruct((_N, _D), jnp.float32),
    )(x, W)


def _sc_scatter(m, idx, zeros_zb):
    mesh = plsc.VectorSubcoreMesh(core_axis_name="c", subcore_axis_name="s")

    @functools.partial(
        pl.kernel,
        mesh=mesh,
        out_type=jax.ShapeDtypeStruct((_NC, _NPAD, _D), jnp.float32),
        scratch_types=(
            [pltpu.VMEM((2 * _NSL, _CH), jnp.int32)] * 2
            + [pltpu.VMEM((_CH, _D), jnp.float32)] * _NSL
            + [pltpu.VMEM_SHARED((_NPAD, _D), jnp.float32)]
            + [pltpu.SemaphoreType.DMA] * (2 * _NSL + 2)
        ),
    )
    def k(m_hbm, idx_hbm, z_hbm, out_hbm, *refs):
        gidx = refs[0:2]
        bufs = refs[2:2 + _NSL]
        agg_sh = refs[2 + _NSL]
        gsems = refs[3 + _NSL:3 + 2 * _NSL]
        ssems = refs[3 + 2 * _NSL:3 + 3 * _NSL]
        isems = refs[3 + 3 * _NSL:5 + 3 * _NSL]
        c = lax.axis_index("c")
        s = lax.axis_index("s")
        w = c * _NS + s
        pltpu.sync_copy(z_hbm, bufs[0].at[pl.ds(0, _ZB)])
        row0 = s * _RPT
        for j in range(_RPT // _ZB):
            pltpu.sync_copy(bufs[0].at[pl.ds(0, _ZB)],
                            agg_sh.at[pl.ds(row0 + j * _ZB, _ZB), :])
        plsc.subcore_barrier()

        def idx_wait(p):
            pltpu.make_async_copy(idx_hbm.at[w, 0], gidx[p], isems[p]).wait()

        def phase(g_dyn, p, nxt_g_dyn, prefetch, drain_only):
            for t in range(_NSL):
                pltpu.make_async_copy(
                    m_hbm.at[gidx[p].at[0]], bufs[t], gsems[t]).wait()
                pltpu.async_copy(
                    bufs[t], agg_sh.at[gidx[p].at[2 * t + 1]], ssems[t],
                    add=True)
            if not drain_only:
                idx_wait(1 - p)
            for t in range(_NSL):
                pltpu.make_async_copy(
                    bufs[t], agg_sh.at[gidx[p].at[2 * t + 1]],
                    ssems[t]).wait()
                if not drain_only:
                    pltpu.async_copy(
                        m_hbm.at[gidx[1 - p].at[2 * t]], bufs[t], gsems[t])
            if prefetch:
                pltpu.async_copy(idx_hbm.at[w, nxt_g_dyn], gidx[p], isems[p])

        pltpu.sync_copy(idx_hbm.at[w, 0], gidx[0])
        pltpu.async_copy(idx_hbm.at[w, 1], gidx[1], isems[1])
        for t in range(_NSL):
            pltpu.async_copy(m_hbm.at[gidx[0].at[2 * t]], bufs[t], gsems[t])

        def body(kk, carry):
            phase(2 * kk, 0, 2 * kk + 2, True, False)
            phase(2 * kk + 1, 1, 2 * kk + 3, True, False)
            return carry

        lax.fori_loop(0, _NGR // 2 - 1, body, 0)
        phase(_NGR - 2, 0, 0, False, False)
        phase(_NGR - 1, 1, 0, False, True)
        plsc.subcore_barrier()

        for j in range(_RPT // _ZB):
            r0 = row0 + j * _ZB
            pltpu.sync_copy(agg_sh.at[pl.ds(r0, _ZB), :], bufs[0].at[pl.ds(0, _ZB)])
            pltpu.sync_copy(bufs[0].at[pl.ds(0, _ZB)],
                            out_hbm.at[c, pl.ds(r0, _ZB), :])

    return k(m, idx, zeros_zb)


def _gru(agg0_ref, agg1_ref, x_ref, wih_ref, bih_ref, whh_ref, bhh_ref):
    agg = agg0_ref[0] + agg1_ref[0]
    x = x_ref[...]
    gi = lax.dot_general(
        agg, wih_ref[...], (((1,), (1,)), ((), ())),
        preferred_element_type=jnp.float32) + bih_ref[...]
    gh = lax.dot_general(
        x, whh_ref[...], (((1,), (1,)), ((), ())),
        preferred_element_type=jnp.float32) + bhh_ref[...]
    r = jax.nn.sigmoid(gi[:, 0:_D] + gh[:, 0:_D])
    z = jax.nn.sigmoid(gi[:, _D:2 * _D] + gh[:, _D:2 * _D])
    n = jnp.tanh(gi[:, 2 * _D:] + r * gh[:, 2 * _D:])
    return jnp.maximum((1.0 - z) * n + z * x, 0.0)


def _post_body(agg0_ref, agg1_ref, x_ref, wih_ref, bih_ref,
               whh_ref, bhh_ref, o_ref):
    o_ref[...] = _gru(agg0_ref, agg1_ref, x_ref, wih_ref, bih_ref,
                      whh_ref, bhh_ref)


def _fused_body(agg0_ref, agg1_ref, x_ref, wih_ref, bih_ref,
                whh_ref, bhh_ref, wn_ref, o_ref, mo_ref):
    xn = _gru(agg0_ref, agg1_ref, x_ref, wih_ref, bih_ref, whh_ref, bhh_ref)
    o_ref[...] = xn
    mo_ref[...] = jnp.dot(xn, wn_ref[...], preferred_element_type=jnp.float32)


_POST_SPECS = [
    pl.BlockSpec((1, _RB, _D), lambda i: (0, i, 0)),
    pl.BlockSpec((1, _RB, _D), lambda i: (1, i, 0)),
    pl.BlockSpec((_RB, _D), lambda i: (i, 0)),
    pl.BlockSpec((_H3, _D), lambda i: (0, 0)),
    pl.BlockSpec((1, _H3), lambda i: (0, 0)),
    pl.BlockSpec((_H3, _D), lambda i: (0, 0)),
    pl.BlockSpec((1, _H3), lambda i: (0, 0)),
]


def _post(aggout, x, Wih, bih, Whh, bhh):
    return pl.pallas_call(
        _post_body,
        grid=(_NBLK,),
        in_specs=_POST_SPECS,
        out_specs=pl.BlockSpec((_RB, _D), lambda i: (i, 0)),
        out_shape=jax.ShapeDtypeStruct((_N, _D), jnp.float32),
    )(aggout, aggout, x, Wih, bih.reshape(1, _H3), Whh, bhh.reshape(1, _H3))


def _fused(aggout, x, Wih, bih, Whh, bhh, Wn):
    return pl.pallas_call(
        _fused_body,
        grid=(_NBLK,),
        in_specs=_POST_SPECS + [pl.BlockSpec((_D, _D), lambda i: (0, 0))],
        out_specs=[
            pl.BlockSpec((_RB, _D), lambda i: (i, 0)),
            pl.BlockSpec((_RB, _D), lambda i: (i, 0)),
        ],
        out_shape=[
            jax.ShapeDtypeStruct((_N, _D), jnp.float32),
            jax.ShapeDtypeStruct((_N, _D), jnp.float32),
        ],
    )(aggout, aggout, x, Wih, bih.reshape(1, _H3), Whh, bhh.reshape(1, _H3),
      Wn)


def _pool_body(x_ref, b_ref, fc1w_ref, fc1b_ref, fc2w_ref, fc2b_ref,
               o_ref, sums_ref, cnts_ref):
    i = pl.program_id(0)

    @pl.when(i == 0)
    def _():
        sums_ref[...] = jnp.zeros_like(sums_ref)
        cnts_ref[...] = jnp.zeros_like(cnts_ref)

    x = x_ref[...]
    b = b_ref[...]
    gids = lax.broadcasted_iota(jnp.int32, (_RB, _G), 1)
    onehot = (b == gids).astype(jnp.float32)
    sums_ref[...] += lax.dot_general(
        onehot, x, (((0,), (0,)), ((), ())),
        preferred_element_type=jnp.float32)
    cnts_ref[...] += lax.dot_general(
        onehot, jnp.ones((_RB, _G), jnp.float32), (((0,), (0,)), ((), ())),
        preferred_element_type=jnp.float32)

    @pl.when(i == _NBLK - 1)
    def _():
        hg = sums_ref[...] / jnp.maximum(cnts_ref[...], 1.0)
        hg = jnp.dot(hg, fc1w_ref[...],
                     preferred_element_type=jnp.float32) + fc1b_ref[...]
        hg = jnp.where(hg > 0, hg, jnp.exp(hg) - 1.0)
        hg = jnp.dot(hg, fc2w_ref[...],
                     preferred_element_type=jnp.float32) + fc2b_ref[...]
        mx = jnp.max(hg, axis=0, keepdims=True)
        lse = jnp.log(jnp.sum(jnp.exp(hg - mx), axis=0, keepdims=True))
        o_ref[...] = hg - mx - lse


def _pool(x, batch2d, fc1_w, fc1_b, fc2_w, fc2_b):
    return pl.pallas_call(
        _pool_body,
        grid=(_NBLK,),
        in_specs=[
            pl.BlockSpec((_RB, _D), lambda i: (i, 0)),
            pl.BlockSpec((_RB, 1), lambda i: (i, 0)),
            pl.BlockSpec((_D, _D), lambda i: (0, 0)),
            pl.BlockSpec((1, _D), lambda i: (0, 0)),
            pl.BlockSpec((_D, _C), lambda i: (0, 0)),
            pl.BlockSpec((1, _C), lambda i: (0, 0)),
        ],
        out_specs=pl.BlockSpec((_G, _C), lambda i: (0, 0)),
        out_shape=jax.ShapeDtypeStruct((_G, _C), jnp.float32),
        scratch_shapes=[
            pltpu.VMEM((_G, _G), jnp.float32),
            pltpu.VMEM((_G, _G), jnp.float32),
        ],
    )(x, batch2d, fc1_w, fc1_b.reshape(1, _D), fc2_w, fc2_b.reshape(1, _C))


def kernel(h, edge_index, edge_attr, batch,
           W0, Wih0, Whh0, bih0, bhh0,
           W1, Wih1, Whh1, bih1, bhh1,
           W2, Wih2, Whh2, bih2, bhh2,
           W3, Wih3, Whh3, bih3, bhh3,
           fc1_w, fc1_b, fc2_w, fc2_b):
    pad = _NW * _CPT * _CH - _E
    ar = jnp.arange(pad, dtype=jnp.int32)
    src = jnp.concatenate([edge_index[0], ar * 13 % _N])
    dst = jnp.concatenate([edge_index[1], _N + ar % (_NPAD - _N)])
    idx = jnp.stack(
        [src.reshape(_NW, _NGR, _NSL, _CH),
         dst.reshape(_NW, _NGR, _NSL, _CH)], axis=3
    ).reshape(_NW, _NGR, 2 * _NSL, _CH)
    zeros_zb = jnp.zeros((_ZB, _D), jnp.float32)
    params = [
        (W0, Wih0, Whh0, bih0, bhh0),
        (W1, Wih1, Whh1, bih1, bhh1),
        (W2, Wih2, Whh2, bih2, bhh2),
        (W3, Wih3, Whh3, bih3, bhh3),
    ]
    x = h
    m = _pre(x, W0)
    for li, (W, Wih, Whh, bih, bhh) in enumerate(params):
        aggout = _sc_scatter(m, idx, zeros_zb)
        if li < 3:
            x, m = _fused(aggout, x, Wih, bih, Whh, bhh, params[li + 1][0])
        else:
            x = _post(aggout, x, Wih, bih, Whh, bhh)
    return _pool(x, batch.reshape(_N, 1), fc1_w, fc1_b, fc2_w, fc2_b)


# speedup vs baseline: 1.0265x; 1.0265x over previous
import functools

import jax
import jax.numpy as jnp
from jax import lax
from jax.experimental import pallas as pl
from jax.experimental.pallas import tpu as pltpu
from jax.experimental.pallas import tpu_sc as plsc

_N = 10000
_E = 320000
_D = 128
_G = 128
_C = 10
_H3 = 3 * _D

_NC, _NS = 2, 16
_NW = _NC * _NS
_CH = 80
_CPT = 128
_NSL = 4
_NGR = _CPT // _NSL
_NPAD = 10240
_RPT = _NPAD // _NS
_ZB = 80

_RB = 1000
_NBLK = _N // _RB


def _pre_body(x_ref, w_ref, m_ref):
    m_ref[...] = jnp.dot(x_ref[...], w_ref[...],
                         preferred_element_type=jnp.float32)


def _pre(x, W):
    return pl.pallas_call(
        _pre_body,
        grid=(_NBLK,),
        in_specs=[
            pl.BlockSpec((_RB, _D), lambda i: (i, 0)),
            pl.BlockSpec((_D, _D), lambda i: (0, 0)),
        ],
        out_specs=pl.BlockSpec((_RB, _D), lambda i: (i, 0)),
        out_shape=jax.ShapeDtypeStruct((_N, _D), jnp.float32),
    )(x, W)


def _sc_scatter(m, idx, zeros_zb):
    mesh = plsc.VectorSubcoreMesh(core_axis_name="c", subcore_axis_name="s")

    @functools.partial(
        pl.kernel,
        mesh=mesh,
        out_type=jax.ShapeDtypeStruct((_NC, _NPAD, _D), jnp.float32),
        scratch_types=(
            [pltpu.VMEM((2 * _NSL, _CH), jnp.int32)] * 2
            + [pltpu.VMEM((_CH, _D), jnp.float32)] * _NSL
            + [pltpu.VMEM_SHARED((_NPAD, _D), jnp.float32)]
            + [pltpu.SemaphoreType.DMA] * (2 * _NSL + 2)
        ),
    )
    def k(m_hbm, idx_hbm, z_hbm, out_hbm, *refs):
        gidx = refs[0:2]
        bufs = refs[2:2 + _NSL]
        agg_sh = refs[2 + _NSL]
        gsems = refs[3 + _NSL:3 + 2 * _NSL]
        ssems = refs[3 + 2 * _NSL:3 + 3 * _NSL]
        isems = refs[3 + 3 * _NSL:5 + 3 * _NSL]
        c = lax.axis_index("c")
        s = lax.axis_index("s")
        w = c * _NS + s
        pltpu.sync_copy(z_hbm, bufs[0].at[pl.ds(0, _ZB)])
        row0 = s * _RPT
        for j in range(_RPT // _ZB):
            pltpu.sync_copy(bufs[0].at[pl.ds(0, _ZB)],
                            agg_sh.at[pl.ds(row0 + j * _ZB, _ZB), :])
        plsc.subcore_barrier()

        def idx_wait(p):
            pltpu.make_async_copy(idx_hbm.at[w, 0], gidx[p], isems[p]).wait()

        def phase(g_dyn, p, nxt_g_dyn, prefetch, drain_only):
            for t in range(_NSL):
                pltpu.make_async_copy(
                    m_hbm.at[gidx[p].at[0]], bufs[t], gsems[t]).wait()
                pltpu.async_copy(
                    bufs[t], agg_sh.at[gidx[p].at[2 * t + 1]], ssems[t],
                    add=True)
            if not drain_only:
                idx_wait(1 - p)
            for t in range(_NSL):
                pltpu.make_async_copy(
                    bufs[t], agg_sh.at[gidx[p].at[2 * t + 1]],
                    ssems[t]).wait()
                if not drain_only:
                    pltpu.async_copy(
                        m_hbm.at[gidx[1 - p].at[2 * t]], bufs[t], gsems[t])
            if prefetch:
                pltpu.async_copy(idx_hbm.at[w, nxt_g_dyn], gidx[p], isems[p])

        pltpu.sync_copy(idx_hbm.at[w, 0], gidx[0])
        pltpu.async_copy(idx_hbm.at[w, 1], gidx[1], isems[1])
        for t in range(_NSL):
            pltpu.async_copy(m_hbm.at[gidx[0].at[2 * t]], bufs[t], gsems[t])

        def body(kk, carry):
            phase(2 * kk, 0, 2 * kk + 2, True, False)
            phase(2 * kk + 1, 1, 2 * kk + 3, True, False)
            return carry

        lax.fori_loop(0, _NGR // 2 - 1, body, 0)
        phase(_NGR - 2, 0, 0, False, False)
        phase(_NGR - 1, 1, 0, False, True)
        plsc.subcore_barrier()

        for j in range(_RPT // _ZB):
            r0 = row0 + j * _ZB
            pltpu.sync_copy(agg_sh.at[pl.ds(r0, _ZB), :], bufs[0].at[pl.ds(0, _ZB)])
            pltpu.sync_copy(bufs[0].at[pl.ds(0, _ZB)],
                            out_hbm.at[c, pl.ds(r0, _ZB), :])

    return k(m, idx, zeros_zb)


def _gru(agg0_ref, agg1_ref, x_ref, wih_ref, bih_ref, whh_ref, bhh_ref):
    agg = agg0_ref[0] + agg1_ref[0]
    x = x_ref[...]
    gi = lax.dot_general(
        agg, wih_ref[...], (((1,), (1,)), ((), ())),
        preferred_element_type=jnp.float32) + bih_ref[...]
    gh = lax.dot_general(
        x, whh_ref[...], (((1,), (1,)), ((), ())),
        preferred_element_type=jnp.float32) + bhh_ref[...]
    r = jax.nn.sigmoid(gi[:, 0:_D] + gh[:, 0:_D])
    z = jax.nn.sigmoid(gi[:, _D:2 * _D] + gh[:, _D:2 * _D])
    n = jnp.tanh(gi[:, 2 * _D:] + r * gh[:, 2 * _D:])
    return jnp.maximum((1.0 - z) * n + z * x, 0.0)


def _post_body(agg0_ref, agg1_ref, x_ref, wih_ref, bih_ref,
               whh_ref, bhh_ref, o_ref):
    o_ref[...] = _gru(agg0_ref, agg1_ref, x_ref, wih_ref, bih_ref,
                      whh_ref, bhh_ref)


def _fused_body(agg0_ref, agg1_ref, x_ref, wih_ref, bih_ref,
                whh_ref, bhh_ref, wn_ref, o_ref, mo_ref):
    xn = _gru(agg0_ref, agg1_ref, x_ref, wih_ref, bih_ref, whh_ref, bhh_ref)
    o_ref[...] = xn
    mo_ref[...] = jnp.dot(xn, wn_ref[...], preferred_element_type=jnp.float32)


_POST_SPECS = [
    pl.BlockSpec((1, _RB, _D), lambda i: (0, i, 0)),
    pl.BlockSpec((1, _RB, _D), lambda i: (1, i, 0)),
    pl.BlockSpec((_RB, _D), lambda i: (i, 0)),
    pl.BlockSpec((_H3, _D), lambda i: (0, 0)),
    pl.BlockSpec((1, _H3), lambda i: (0, 0)),
    pl.BlockSpec((_H3, _D), lambda i: (0, 0)),
    pl.BlockSpec((1, _H3), lambda i: (0, 0)),
]


def _post(aggout, x, Wih, bih, Whh, bhh):
    return pl.pallas_call(
        _post_body,
        grid=(_NBLK,),
        in_specs=_POST_SPECS,
        out_specs=pl.BlockSpec((_RB, _D), lambda i: (i, 0)),
        out_shape=jax.ShapeDtypeStruct((_N, _D), jnp.float32),
    )(aggout, aggout, x, Wih, bih.reshape(1, _H3), Whh, bhh.reshape(1, _H3))


def _fused(aggout, x, Wih, bih, Whh, bhh, Wn):
    return pl.pallas_call(
        _fused_body,
        grid=(_NBLK,),
        in_specs=_POST_SPECS + [pl.BlockSpec((_D, _D), lambda i: (0, 0))],
        out_specs=[
            pl.BlockSpec((_RB, _D), lambda i: (i, 0)),
            pl.BlockSpec((_RB, _D), lambda i: (i, 0)),
        ],
        out_shape=[
            jax.ShapeDtypeStruct((_N, _D), jnp.float32),
            jax.ShapeDtypeStruct((_N, _D), jnp.float32),
        ],
    )(aggout, aggout, x, Wih, bih.reshape(1, _H3), Whh, bhh.reshape(1, _H3),
      Wn)


def _pool_body(x_ref, b_ref, fc1w_ref, fc1b_ref, fc2w_ref, fc2b_ref,
               o_ref, sums_ref, cnts_ref):
    i = pl.program_id(0)

    @pl.when(i == 0)
    def _():
        sums_ref[...] = jnp.zeros_like(sums_ref)
        cnts_ref[...] = jnp.zeros_like(cnts_ref)

    x = x_ref[...]
    b = b_ref[...]
    gids = lax.broadcasted_iota(jnp.int32, (_RB, _G), 1)
    onehot = (b == gids).astype(jnp.float32)
    sums_ref[...] += lax.dot_general(
        onehot, x, (((0,), (0,)), ((), ())),
        preferred_element_type=jnp.float32)
    cnts_ref[...] += lax.dot_general(
        onehot, jnp.ones((_RB, _G), jnp.float32), (((0,), (0,)), ((), ())),
        preferred_element_type=jnp.float32)

    @pl.when(i == _NBLK - 1)
    def _():
        hg = sums_ref[...] / jnp.maximum(cnts_ref[...], 1.0)
        hg = jnp.dot(hg, fc1w_ref[...],
                     preferred_element_type=jnp.float32) + fc1b_ref[...]
        hg = jnp.where(hg > 0, hg, jnp.exp(hg) - 1.0)
        hg = jnp.dot(hg, fc2w_ref[...],
                     preferred_element_type=jnp.float32) + fc2b_ref[...]
        mx = jnp.max(hg, axis=0, keepdims=True)
        lse = jnp.log(jnp.sum(jnp.exp(hg - mx), axis=0, keepdims=True))
        o_ref[...] = hg - mx - lse


def _pool(x, batch2d, fc1_w, fc1_b, fc2_w, fc2_b):
    return pl.pallas_call(
        _pool_body,
        grid=(_NBLK,),
        in_specs=[
            pl.BlockSpec((_RB, _D), lambda i: (i, 0)),
            pl.BlockSpec((_RB, 1), lambda i: (i, 0)),
            pl.BlockSpec((_D, _D), lambda i: (0, 0)),
            pl.BlockSpec((1, _D), lambda i: (0, 0)),
            pl.BlockSpec((_D, _C), lambda i: (0, 0)),
            pl.BlockSpec((1, _C), lambda i: (0, 0)),
        ],
        out_specs=pl.BlockSpec((_G, _C), lambda i: (0, 0)),
        out_shape=jax.ShapeDtypeStruct((_G, _C), jnp.float32),
        scratch_shapes=[
            pltpu.VMEM((_G, _G), jnp.float32),
            pltpu.VMEM((_G, _G), jnp.float32),
        ],
    )(x, batch2d, fc1_w, fc1_b.reshape(1, _D), fc2_w, fc2_b.reshape(1, _C))


def kernel(h, edge_index, edge_attr, batch,
           W0, Wih0, Whh0, bih0, bhh0,
           W1, Wih1, Whh1, bih1, bhh1,
           W2, Wih2, Whh2, bih2, bhh2,
           W3, Wih3, Whh3, bih3, bhh3,
           fc1_w, fc1_b, fc2_w, fc2_b):
    pad = _NW * _CPT * _CH - _E
    ar = jnp.arange(pad, dtype=jnp.int32)
    src = jnp.concatenate([edge_index[0], ar * 13 % _N])
    dst = jnp.concatenate([edge_index[1], _N + ar % (_NPAD - _N)])
    idx = jnp.stack(
        [src.reshape(_NW, _NGR, _NSL, _CH),
         dst.reshape(_NW, _NGR, _NSL, _CH)], axis=3
    ).reshape(_NW, _NGR, 2 * _NSL, _CH)
    zeros_zb = jnp.zeros((_ZB, _D), jnp.float32)
    params = [
        (W0, Wih0, Whh0, bih0, bhh0),
        (W1, Wih1, Whh1, bih1, bhh1),
        (W2, Wih2, Whh2, bih2, bhh2),
        (W3, Wih3, Whh3, bih3, bhh3),
    ]
    x = h
    m = _pre(x, W0)
    for li, (W, Wih, Whh, bih, bhh) in enumerate(params):
        aggout = _sc_scatter(m, idx, zeros_zb)
        if li < 3:
            x, m = _fused(aggout, x, Wih, bih, Whh, bhh, params[li + 1][0])
        else:
            x = _post(aggout, x, Wih, bih, Whh, bhh)
    return _pool(x, batch.reshape(_N, 1), fc1_w, fc1_b, fc2_w, fc2_b)
